# baseline (speedup 1.0000x reference)
import functools

import jax
import jax.numpy as jnp
from jax import lax
from jax.experimental import pallas as pl
from jax.experimental.pallas import tpu as pltpu
from jax.experimental.pallas import tpu_sc as plsc

N = 10000
NP = 10240
H = 128
E = 320000
NC = 2
NS = 16
NW = NC * NS
EPW = E // NW
CH = 80
NCH = EPW // CH
CHP = 80
NCHP = 125
EPWP = CHP * NCHP
EP = EPWP * NW
RPT = NP // NS
CW = 128
CWT = 8

@functools.cache
def _make_seg_sum():
    mesh = plsc.VectorSubcoreMesh(core_axis_name="c", subcore_axis_name="s",
                                  num_cores=NC, num_subcores=NS)
    return pl.kernel(
        _seg_sum_body,
        out_type=jax.ShapeDtypeStruct((NC, NP, H), jnp.float32),
        mesh=mesh,
        scratch_types=[
            pltpu.VMEM((CHP,), jnp.int32),
            pltpu.VMEM((CHP,), jnp.int32),
            pltpu.VMEM((CHP,), jnp.int32),
            pltpu.VMEM((CHP,), jnp.int32),
            pltpu.VMEM((CHP,), jnp.int32),
            pltpu.VMEM((CHP,), jnp.int32),
            pltpu.VMEM((CHP,), jnp.int32),
            pltpu.VMEM((CHP,), jnp.int32),
            pltpu.VMEM((CHP, H), jnp.float32),
            pltpu.VMEM((CHP, H), jnp.float32),
            pltpu.VMEM((CHP, H), jnp.float32),
            pltpu.VMEM_SHARED((NP, H), jnp.float32),
            pltpu.SemaphoreType.DMA,
            pltpu.SemaphoreType.DMA,
        ],
    )


def _seg_sum(h, src, dst):
    return _make_seg_sum()(h, src, dst)


def _seg_sum_body(h_hbm, src_hbm, dst_hbm, out_hbm, si0, si1, si2, si3,
                  dc0, dc1, dc2, dc3, rows0, rows1, rows2, acc, gsem, isem):
    sis = (si0, si1, si2, si3)
    dcs = (dc0, dc1, dc2, dc3)
    rows = (rows0, rows1, rows2)
    c = lax.axis_index("c")
    s = lax.axis_index("s")
    wid = s * NC + c
    ebase = wid * EPWP

    zero = jnp.zeros((16,), jnp.float32)

    def zrow(i, carry):
        for j in range(H // 16):
            rows0[i, pl.ds(j * 16, 16)] = zero
        return carry

    lax.fori_loop(0, CHP, zrow, 0)
    for t in range(RPT // CHP):
        pltpu.sync_copy(rows0, acc.at[pl.ds(s * RPT + t * CHP, CHP)])
    rem = RPT % CHP
    if rem:
        pltpu.sync_copy(rows0.at[pl.ds(0, rem)],
                        acc.at[pl.ds(s * RPT + (RPT // CHP) * CHP, rem)])
    plsc.subcore_barrier()

    def fire_idx(ch, b):
        pltpu.async_copy(src_hbm.at[pl.ds(ebase + ch * CHP, CHP)], sis[b], isem)
        pltpu.async_copy(dst_hbm.at[pl.ds(ebase + ch * CHP, CHP)], dcs[b], isem)

    def wait_idx(b):
        pltpu.make_async_copy(src_hbm.at[pl.ds(0, CHP)], sis[b], isem).wait()
        pltpu.make_async_copy(dst_hbm.at[pl.ds(0, CHP)], dcs[b], isem).wait()

    def fire_gather(r, b):
        pltpu.async_copy(h_hbm.at[sis[b]], rows[r], gsem)

    def wait_gather(r):
        pltpu.make_async_copy(h_hbm.at[sis[0]], rows[r], gsem).wait()

    def scat(r, b):
        pltpu.sync_copy(rows[r], acc.at[dcs[b]], add=True)

    def slot(ch, j):
        if isinstance(ch, int):
            has_gather = ch + 2 < NCHP
            has_idx = ch + 3 < NCHP
        else:
            has_gather = has_idx = True
        if has_gather:
            wait_idx((j + 2) % 4)
            fire_gather((j + 2) % 3, (j + 2) % 4)
        wait_gather(j % 3)
        if has_idx:
            fire_idx(ch + 3, (j + 3) % 4)
        scat(j % 3, j % 4)

    fire_idx(0, 0)
    fire_idx(1, 1)
    fire_idx(2, 2)
    wait_idx(0)
    fire_gather(0, 0)
    wait_idx(1)
    fire_gather(1, 1)

    def body(i, carry):
        ch = 12 * i
        for j in range(12):
            slot(ch + j, j)
        return carry

    K = (NCHP - 3) // 12
    lax.fori_loop(0, K, body, 0)
    for ch in range(12 * K, NCHP):
        slot(ch, ch % 12)

    plsc.subcore_barrier()
    pltpu.sync_copy(acc.at[pl.ds(s * RPT, RPT)], out_hbm.at[c, pl.ds(s * RPT, RPT)])


@functools.cache
def _make_seg_count():
    mesh = plsc.VectorSubcoreMesh(core_axis_name="c", subcore_axis_name="s",
                                  num_cores=NC, num_subcores=NS)
    return pl.kernel(
        _seg_count_body,
        out_type=jax.ShapeDtypeStruct((NC, NP, CW), jnp.float32),
        mesh=mesh,
        scratch_types=[
            pltpu.VMEM((EPW,), jnp.int32),
            pltpu.VMEM((CH,), jnp.int32),
            pltpu.VMEM((CH,), jnp.int32),
            pltpu.VMEM((CH, CW), jnp.float32),
            pltpu.VMEM_SHARED((NP, CW), jnp.float32),
            pltpu.SemaphoreType.DMA,
        ],
    )


def _seg_count(dst):
    ones = jnp.ones((CH, CW), jnp.float32)
    zeros = jnp.zeros((RPT, CW), jnp.float32)
    return _make_seg_count()(dst, ones, zeros)


def _seg_count_body(dst_hbm, ones_hbm, zeros_hbm, out_hbm, didx, dc0, dc1,
                    ones, acc, sem):
    c = lax.axis_index("c")
    s = lax.axis_index("s")
    wid = s * NC + c

    pltpu.sync_copy(dst_hbm.at[pl.ds(wid * EPW, EPW)], didx)
    pltpu.sync_copy(ones_hbm, ones)
    pltpu.sync_copy(zeros_hbm, acc.at[pl.ds(s * RPT, RPT)])
    plsc.subcore_barrier()

    def stage(ch, dc):
        for j in range(CH // 16):
            dc[pl.ds(j * 16, 16)] = didx[pl.ds(ch * CH + j * 16, 16)]

    def fire(dc):
        pltpu.async_copy(ones, acc.at[dc], sem, add=True)

    def wait_one(dc):
        pltpu.make_async_copy(ones, acc.at[dc], sem).wait()

    stage(0, dc0)
    fire(dc0)

    def body(i, carry):
        ch = 2 * i
        stage(ch + 1, dc1)
        fire(dc1)
        wait_one(dc0)
        stage(ch + 2, dc0)
        fire(dc0)
        wait_one(dc1)
        return carry

    lax.fori_loop(0, (NCH - 1) // 2, body, 0)
    wait_one(dc0)
    plsc.subcore_barrier()
    pltpu.sync_copy(acc.at[pl.ds(s * RPT, RPT)], out_hbm.at[c, pl.ds(s * RPT, RPT)])


_BR = 1280


def _dot(a, b):
    return jnp.dot(a, b, preferred_element_type=jnp.float32,
                   precision=lax.Precision.HIGHEST)


def _row_spec():
    return pl.BlockSpec((_BR, H), lambda i: (i, 0))


def _cnt_spec():
    return pl.BlockSpec((_BR, CWT), lambda i: (i, 0))


def _w_spec():
    return pl.BlockSpec((H, H), lambda i: (0, 0))


def _b_spec():
    return pl.BlockSpec((1, H), lambda i: (0, 0))


def _in_proj_body(xu, au, cu, xi, ai, ci, ou, oi):
    ou[...] = jnp.maximum(_dot(xu[...], au[...]) + cu[...], 0.0)
    oi[...] = jnp.maximum(_dot(xi[...], ai[...]) + ci[...], 0.0)


def _in_proj2(xu, au, cu, xi, ai, ci):
    return pl.pallas_call(
        _in_proj_body,
        grid=(NP // _BR,),
        in_specs=[_row_spec(), _w_spec(), _b_spec(),
                  _row_spec(), _w_spec(), _b_spec()],
        out_specs=[_row_spec(), _row_spec()],
        out_shape=[jax.ShapeDtypeStruct((NP, H), jnp.float32)] * 2,
    )(xu, au, cu, xi, ai, ci)


def _sage_update(a0, a1, cnt, h, al, ar, cv):
    inv = 1.0 / jnp.maximum(cnt[...][:, :1], 1.0)
    mean = (a0[...] + a1[...]) * inv
    hh = h[...]
    z = _dot(mean, al[...]) + _dot(hh, ar[...]) + cv[...]
    return jnp.maximum(z, 0.0) + hh


def _layer_mid_body(a0i, a1i, ci, hi, ali, ari, cvi,
                    a0u, a1u, cu, hu, alu, aru, cvu, oi, ou):
    oi[...] = _sage_update(a0i, a1i, ci, hi, ali, ari, cvi)
    ou[...] = _sage_update(a0u, a1u, cu, hu, alu, aru, cvu)


def _layer_fin_body(a0i, a1i, ci, hi, ali, ari, cvi,
                    a0u, a1u, cu, hu, alu, aru, cvu, wf, bf, oi, ou):
    ni = _sage_update(a0i, a1i, ci, hi, ali, ari, cvi)
    nu = _sage_update(a0u, a1u, cu, hu, alu, aru, cvu)
    oi[...] = _dot(ni, wf[...]) + bf[...]
    ou[...] = _dot(nu, wf[...]) + bf[...]


def _layer2(item_args, user_args, fin=None):
    half = [_row_spec(), _row_spec(), _cnt_spec(), _row_spec(),
            _w_spec(), _w_spec(), _b_spec()]
    in_specs = half + half
    args = list(item_args) + list(user_args)
    if fin is None:
        body = _layer_mid_body
    else:
        body = _layer_fin_body
        in_specs = in_specs + [_w_spec(), _b_spec()]
        args = args + list(fin)
    return pl.pallas_call(
        body,
        grid=(NP // _BR,),
        in_specs=in_specs,
        out_specs=[_row_spec(), _row_spec()],
        out_shape=[jax.ShapeDtypeStruct((NP, H), jnp.float32)] * 2,
    )(*args)


_BN_S = 1.0 / jnp.sqrt(jnp.float32(1.0 + 1e-5))


def kernel(x_user, x_item, edge_index_user_to_item, edge_index_item_to_user,
           params):
    src_ui = edge_index_user_to_item[0]
    dst_ui = edge_index_user_to_item[1]
    src_iu = edge_index_item_to_user[0]
    dst_iu = edge_index_item_to_user[1]

    def pad_edges(src, dst):
        padw = ((0, 0), (0, EPWP - EPW))
        src_p = jnp.pad(src.reshape(NW, EPW), padw).reshape(EP)
        dst_p = jnp.pad(dst.reshape(NW, EPW), padw,
                        constant_values=NP - 1).reshape(EP)
        return src_p, dst_p

    src_ui_p, dst_ui_p = pad_edges(src_ui, dst_ui)
    src_iu_p, dst_iu_p = pad_edges(src_iu, dst_iu)

    pad = ((0, NP - N), (0, 0))
    xs = {"user": jnp.pad(x_user, pad), "item": jnp.pad(x_item, pad)}

    cnt_item = _seg_count(dst_ui)
    cnt_user = _seg_count(dst_iu)
    cnt_item = (cnt_item[0] + cnt_item[1])[:, :CWT]
    cnt_user = (cnt_user[0] + cnt_user[1])[:, :CWT]
    cnt = {"item": cnt_item, "user": cnt_user}

    def folded(W, b, w2, b2):
        s = w2 * _BN_S
        return W.T * s[None, :], (b * s + b2)[None, :]

    au, cu = folded(*params["lin_in"]["user"], *params["bn_in"]["user"])
    ai, ci = folded(*params["lin_in"]["item"], *params["bn_in"]["item"])
    h_user, h_item = _in_proj2(xs["user"], au, cu, xs["item"], ai, ci)

    Wf, bf = params["final"]
    n_layers = len(params["layers"])
    for li, layer in enumerate(params["layers"]):
        agg_item = _seg_sum(h_user, src_ui_p, dst_ui_p)
        agg_user = _seg_sum(h_item, src_iu_p, dst_iu_p)
        halves = {}
        for nt, agg, conv_key, hh in (("item", agg_item, "user_to_item", h_item),
                                      ("user", agg_user, "item_to_user", h_user)):
            Wl, bl, Wr = layer["conv"][conv_key]
            w2, b2 = layer["bn"][nt]
            s = w2 * _BN_S
            halves[nt] = (agg[0], agg[1], cnt[nt], hh,
                          Wl.T * s[None, :], Wr.T * s[None, :],
                          (bl * s + b2)[None, :])
        fin = (Wf.T, bf[None, :]) if li == n_layers - 1 else None
        out_item, out_user = _layer2(halves["item"], halves["user"], fin)
        h_item, h_user = out_item, out_user

    return (out_user[:N], out_item[:N])


# speedup vs baseline: 1.0564x; 1.0564x over previous
import functools

import jax
import jax.numpy as jnp
from jax import lax
from jax.experimental import pallas as pl
from jax.experimental.pallas import tpu as pltpu
from jax.experimental.pallas import tpu_sc as plsc

N = 10000
NP = 10240
H = 128
E = 320000
NC = 2
NS = 16
NW = NC * NS
EPW = E // NW
CH = 80
NCH = EPW // CH
CHP = 80
NCHP = 125
EPWP = CHP * NCHP
EP = EPWP * NW
RPT = NP // NS
CW = 128
CWT = 8

@functools.cache
def _make_seg_sum():
    mesh = plsc.VectorSubcoreMesh(core_axis_name="c", subcore_axis_name="s",
                                  num_cores=NC, num_subcores=NS)
    return pl.kernel(
        _seg_sum_body,
        out_type=jax.ShapeDtypeStruct((NC, NP, H), jnp.float32),
        mesh=mesh,
        scratch_types=[
            pltpu.VMEM((CHP,), jnp.int32),
            pltpu.VMEM((CHP,), jnp.int32),
            pltpu.VMEM((CHP,), jnp.int32),
            pltpu.VMEM((CHP,), jnp.int32),
            pltpu.VMEM((CHP,), jnp.int32),
            pltpu.VMEM((CHP,), jnp.int32),
            pltpu.VMEM((CHP,), jnp.int32),
            pltpu.VMEM((CHP,), jnp.int32),
            pltpu.VMEM((CHP, H), jnp.float32),
            pltpu.VMEM((CHP, H), jnp.float32),
            pltpu.VMEM((CHP, H), jnp.float32),
            pltpu.VMEM_SHARED((NP, H), jnp.float32),
            pltpu.SemaphoreType.DMA,
            pltpu.SemaphoreType.DMA,
        ],
    )


def _seg_sum(h, src, dst):
    return _make_seg_sum()(h, src, dst)


def _seg_sum_body(h_hbm, src_hbm, dst_hbm, out_hbm, si0, si1, si2, si3,
                  dc0, dc1, dc2, dc3, rows0, rows1, rows2, acc, gsem, isem):
    sis = (si0, si1, si2, si3)
    dcs = (dc0, dc1, dc2, dc3)
    rows = (rows0, rows1, rows2)
    c = lax.axis_index("c")
    s = lax.axis_index("s")
    wid = s * NC + c
    ebase = wid * EPWP

    zero = jnp.zeros((16,), jnp.float32)

    def zrow(i, carry):
        for j in range(H // 16):
            rows0[i, pl.ds(j * 16, 16)] = zero
        return carry

    lax.fori_loop(0, CHP, zrow, 0)
    for t in range(RPT // CHP):
        pltpu.sync_copy(rows0, acc.at[pl.ds(s * RPT + t * CHP, CHP)])
    rem = RPT % CHP
    if rem:
        pltpu.sync_copy(rows0.at[pl.ds(0, rem)],
                        acc.at[pl.ds(s * RPT + (RPT // CHP) * CHP, rem)])
    plsc.subcore_barrier()

    def fire_idx(ch, b):
        pltpu.async_copy(src_hbm.at[pl.ds(ebase + ch * CHP, CHP)], sis[b], isem)
        pltpu.async_copy(dst_hbm.at[pl.ds(ebase + ch * CHP, CHP)], dcs[b], isem)

    def wait_idx(b):
        pltpu.make_async_copy(src_hbm.at[pl.ds(0, CHP)], sis[b], isem).wait()
        pltpu.make_async_copy(dst_hbm.at[pl.ds(0, CHP)], dcs[b], isem).wait()

    def fire_gather(r, b):
        pltpu.async_copy(h_hbm.at[sis[b]], rows[r], gsem)

    def wait_gather(r):
        pltpu.make_async_copy(h_hbm.at[sis[0]], rows[r], gsem).wait()

    def scat(r, b):
        pltpu.sync_copy(rows[r], acc.at[dcs[b]], add=True)

    def slot(ch, j):
        if isinstance(ch, int):
            has_gather = ch + 2 < NCHP
            has_idx = ch + 3 < NCHP
        else:
            has_gather = has_idx = True
        if has_gather:
            wait_idx((j + 2) % 4)
            fire_gather((j + 2) % 3, (j + 2) % 4)
        wait_gather(j % 3)
        if has_idx:
            fire_idx(ch + 3, (j + 3) % 4)
        scat(j % 3, j % 4)

    fire_idx(0, 0)
    fire_idx(1, 1)
    fire_idx(2, 2)
    wait_idx(0)
    fire_gather(0, 0)
    wait_idx(1)
    fire_gather(1, 1)

    def body(i, carry):
        ch = 12 * i
        for j in range(12):
            slot(ch + j, j)
        return carry

    K = (NCHP - 3) // 12
    lax.fori_loop(0, K, body, 0)
    for ch in range(12 * K, NCHP):
        slot(ch, ch % 12)

    plsc.subcore_barrier()
    pltpu.sync_copy(acc.at[pl.ds(s * RPT, RPT)], out_hbm.at[c, pl.ds(s * RPT, RPT)])


@functools.cache
def _make_seg_count():
    mesh = plsc.VectorSubcoreMesh(core_axis_name="c", subcore_axis_name="s",
                                  num_cores=NC, num_subcores=NS)
    return pl.kernel(
        _seg_count_body,
        out_type=jax.ShapeDtypeStruct((NC, NP, CW), jnp.float32),
        mesh=mesh,
        scratch_types=[
            pltpu.VMEM((EPW,), jnp.int32),
            pltpu.VMEM((CH,), jnp.int32),
            pltpu.VMEM((CH,), jnp.int32),
            pltpu.VMEM((CH, CW), jnp.float32),
            pltpu.VMEM_SHARED((NP, CW), jnp.float32),
            pltpu.SemaphoreType.DMA,
        ],
    )


def _seg_count(dst):
    ones = jnp.ones((CH, CW), jnp.float32)
    zeros = jnp.zeros((RPT, CW), jnp.float32)
    return _make_seg_count()(dst, ones, zeros)


def _seg_count_body(dst_hbm, ones_hbm, zeros_hbm, out_hbm, didx, dc0, dc1,
                    ones, acc, sem):
    c = lax.axis_index("c")
    s = lax.axis_index("s")
    wid = s * NC + c

    pltpu.sync_copy(dst_hbm.at[pl.ds(wid * EPW, EPW)], didx)
    pltpu.sync_copy(ones_hbm, ones)
    pltpu.sync_copy(zeros_hbm, acc.at[pl.ds(s * RPT, RPT)])
    plsc.subcore_barrier()

    def stage(ch, dc):
        for j in range(CH // 16):
            dc[pl.ds(j * 16, 16)] = didx[pl.ds(ch * CH + j * 16, 16)]

    def fire(dc):
        pltpu.async_copy(ones, acc.at[dc], sem, add=True)

    def wait_one(dc):
        pltpu.make_async_copy(ones, acc.at[dc], sem).wait()

    stage(0, dc0)
    fire(dc0)

    def body(i, carry):
        ch = 2 * i
        stage(ch + 1, dc1)
        fire(dc1)
        wait_one(dc0)
        stage(ch + 2, dc0)
        fire(dc0)
        wait_one(dc1)
        return carry

    lax.fori_loop(0, (NCH - 1) // 2, body, 0)
    wait_one(dc0)
    plsc.subcore_barrier()
    pltpu.sync_copy(acc.at[pl.ds(s * RPT, RPT)], out_hbm.at[c, pl.ds(s * RPT, RPT)])


_BR = 1280


def _dot(a, b):
    return jnp.dot(a, b, preferred_element_type=jnp.float32,
                   precision=lax.Precision.HIGHEST)


def _row_spec():
    return pl.BlockSpec((_BR, H), lambda i: (i, 0))


def _cnt_spec():
    return pl.BlockSpec((_BR, CWT), lambda i: (i, 0))


def _w_spec():
    return pl.BlockSpec((H, H), lambda i: (0, 0))


def _b_spec():
    return pl.BlockSpec((1, H), lambda i: (0, 0))


def _in_proj_body(xu, au, cu, xi, ai, ci, ou, oi):
    ou[...] = jnp.maximum(_dot(xu[...], au[...]) + cu[...], 0.0)
    oi[...] = jnp.maximum(_dot(xi[...], ai[...]) + ci[...], 0.0)


def _in_proj2(xu, au, cu, xi, ai, ci):
    return pl.pallas_call(
        _in_proj_body,
        grid=(NP // _BR,),
        in_specs=[_row_spec(), _w_spec(), _b_spec(),
                  _row_spec(), _w_spec(), _b_spec()],
        out_specs=[_row_spec(), _row_spec()],
        out_shape=[jax.ShapeDtypeStruct((NP, H), jnp.float32)] * 2,
    )(xu, au, cu, xi, ai, ci)


def _sage_update(a0, a1, cnt, h, al, ar, cv):
    inv = 1.0 / jnp.maximum(cnt[...][:, :1], 1.0)
    mean = (a0[...] + a1[...]) * inv
    hh = h[...]
    z = _dot(mean, al[...]) + _dot(hh, ar[...]) + cv[...]
    return jnp.maximum(z, 0.0) + hh


def _layer_mid_body(a0, a1, cnt, h, al, ar, cv, o):
    o[...] = _sage_update(a0, a1, cnt, h, al, ar, cv)


def _layer_fin_body(a0, a1, cnt, h, al, ar, cv, wf, bf, o):
    o[...] = _dot(_sage_update(a0, a1, cnt, h, al, ar, cv), wf[...]) + bf[...]


def _layer1(args, fin=None):
    in_specs = [_row_spec(), _row_spec(), _cnt_spec(), _row_spec(),
                _w_spec(), _w_spec(), _b_spec()]
    args = list(args)
    if fin is None:
        body = _layer_mid_body
    else:
        body = _layer_fin_body
        in_specs = in_specs + [_w_spec(), _b_spec()]
        args = args + list(fin)
    return pl.pallas_call(
        body,
        grid=(NP // _BR,),
        in_specs=in_specs,
        out_specs=_row_spec(),
        out_shape=jax.ShapeDtypeStruct((NP, H), jnp.float32),
    )(*args)


_BN_S = 1.0 / jnp.sqrt(jnp.float32(1.0 + 1e-5))


def kernel(x_user, x_item, edge_index_user_to_item, edge_index_item_to_user,
           params):
    src_ui = edge_index_user_to_item[0]
    dst_ui = edge_index_user_to_item[1]
    src_iu = edge_index_item_to_user[0]
    dst_iu = edge_index_item_to_user[1]

    def pad_edges(src, dst):
        padw = ((0, 0), (0, EPWP - EPW))
        src_p = jnp.pad(src.reshape(NW, EPW), padw).reshape(EP)
        dst_p = jnp.pad(dst.reshape(NW, EPW), padw,
                        constant_values=NP - 1).reshape(EP)
        return src_p, dst_p

    src_ui_p, dst_ui_p = pad_edges(src_ui, dst_ui)
    src_iu_p, dst_iu_p = pad_edges(src_iu, dst_iu)

    pad = ((0, NP - N), (0, 0))
    xs = {"user": jnp.pad(x_user, pad), "item": jnp.pad(x_item, pad)}

    cnt_item = _seg_count(dst_ui)
    cnt_user = _seg_count(dst_iu)
    cnt_item = (cnt_item[0] + cnt_item[1])[:, :CWT]
    cnt_user = (cnt_user[0] + cnt_user[1])[:, :CWT]
    cnt = {"item": cnt_item, "user": cnt_user}

    def folded(W, b, w2, b2):
        s = w2 * _BN_S
        return W.T * s[None, :], (b * s + b2)[None, :]

    au, cu = folded(*params["lin_in"]["user"], *params["bn_in"]["user"])
    ai, ci = folded(*params["lin_in"]["item"], *params["bn_in"]["item"])
    h_user, h_item = _in_proj2(xs["user"], au, cu, xs["item"], ai, ci)

    Wf, bf = params["final"]
    n_layers = len(params["layers"])
    for li, layer in enumerate(params["layers"]):
        agg_item = _seg_sum(h_user, src_ui_p, dst_ui_p)
        agg_user = _seg_sum(h_item, src_iu_p, dst_iu_p)
        halves = {}
        for nt, agg, conv_key, hh in (("item", agg_item, "user_to_item", h_item),
                                      ("user", agg_user, "item_to_user", h_user)):
            Wl, bl, Wr = layer["conv"][conv_key]
            w2, b2 = layer["bn"][nt]
            s = w2 * _BN_S
            halves[nt] = (agg[0], agg[1], cnt[nt], hh,
                          Wl.T * s[None, :], Wr.T * s[None, :],
                          (bl * s + b2)[None, :])
        fin = (Wf.T, bf[None, :]) if li == n_layers - 1 else None
        out_item = _layer1(halves["item"], fin)
        out_user = _layer1(halves["user"], fin)
        h_item, h_user = out_item, out_user

    return (out_user[:N], out_item[:N])
